# baseline (speedup 1.0000x reference)
import jax
import jax.numpy as jnp
from jax.experimental import pallas as pl
from jax.experimental.pallas import tpu as pltpu

_SWIGLU_SCALE = 1.702
_FB = 256


def _moe_body(x_ref, rw_ref, gate_ref, up_ref, down_ref, out_ref, cw_ref):
    e = pl.program_id(0)
    f = pl.program_id(1)
    x = x_ref[:, 0, :]

    @pl.when((e == 0) & (f == 0))
    def _router():
        logits = jax.lax.dot_general(
            x, rw_ref[...], (((1,), (1,)), ((), ())),
            preferred_element_type=jnp.float32)
        n_e = logits.shape[-1]
        idx = jax.lax.broadcasted_iota(jnp.int32, logits.shape, 1)
        m1 = jnp.max(logits, axis=-1, keepdims=True)
        i1 = jnp.min(jnp.where(logits == m1, idx, n_e), axis=-1, keepdims=True)
        l2 = jnp.where(idx == i1, -jnp.inf, logits)
        m2 = jnp.max(l2, axis=-1, keepdims=True)
        i2 = jnp.min(jnp.where(l2 == m2, idx, n_e), axis=-1, keepdims=True)
        top2 = (idx == i1) | (idx == i2)
        w = jnp.where(top2, jnp.exp(logits - m1), 0.0)
        cw_ref[...] = w / jnp.sum(w, axis=-1, keepdims=True)
        out_ref[:, 0, :] = jnp.zeros(out_ref.shape[::2], out_ref.dtype)

    xb = x.astype(jnp.bfloat16)
    g = jnp.dot(xb, gate_ref[0].astype(jnp.bfloat16),
                preferred_element_type=jnp.float32)
    u = jnp.dot(xb, up_ref[0].astype(jnp.bfloat16),
                preferred_element_type=jnp.float32)
    act = g * jax.nn.sigmoid(_SWIGLU_SCALE * g) * u
    lane = jax.lax.broadcasted_iota(jnp.int32, cw_ref.shape, 1)
    w_e = jnp.sum(jnp.where(lane == e, cw_ref[...], 0.0), axis=-1, keepdims=True)
    out_ref[:, 0, :] += jnp.dot((act * w_e).astype(jnp.bfloat16),
                                down_ref[0].astype(jnp.bfloat16),
                                preferred_element_type=jnp.float32)


def kernel(hidden_states, router_weight, gate_proj, up_proj, down_proj):
    b, s, h = hidden_states.shape
    e, _, f = gate_proj.shape
    nf = f // _FB

    return pl.pallas_call(
        _moe_body,
        grid=(e, nf),
        in_specs=[
            pl.BlockSpec((b, s, h), lambda ei, fi: (0, 0, 0)),
            pl.BlockSpec((e, h), lambda ei, fi: (0, 0)),
            pl.BlockSpec((1, h, _FB), lambda ei, fi: (ei, 0, fi)),
            pl.BlockSpec((1, h, _FB), lambda ei, fi: (ei, 0, fi)),
            pl.BlockSpec((1, _FB, h), lambda ei, fi: (ei, fi, 0)),
        ],
        out_specs=pl.BlockSpec((b, s, h), lambda ei, fi: (0, 0, 0)),
        out_shape=jax.ShapeDtypeStruct((b, s, h), jnp.float32),
        scratch_shapes=[pltpu.VMEM((b * s, e), jnp.float32)],
        compiler_params=pltpu.CompilerParams(
            dimension_semantics=("arbitrary", "arbitrary"),
        ),
    )(hidden_states, router_weight.T, gate_proj, up_proj, down_proj)


# speedup vs baseline: 1.1007x; 1.1007x over previous
import jax
import jax.numpy as jnp
from jax.experimental import pallas as pl
from jax.experimental.pallas import tpu as pltpu

_SWIGLU_SCALE = 1.702
_FB = 512


def _moe_body(x_ref, rw_ref, gate_ref, up_ref, down_ref, out_ref, cw_ref):
    e = pl.program_id(0)
    f = pl.program_id(1)
    x = x_ref[:, 0, :]

    @pl.when((e == 0) & (f == 0))
    def _router():
        logits = jax.lax.dot_general(
            x, rw_ref[...], (((1,), (1,)), ((), ())),
            preferred_element_type=jnp.float32)
        n_e = logits.shape[-1]
        idx = jax.lax.broadcasted_iota(jnp.int32, logits.shape, 1)
        m1 = jnp.max(logits, axis=-1, keepdims=True)
        i1 = jnp.min(jnp.where(logits == m1, idx, n_e), axis=-1, keepdims=True)
        l2 = jnp.where(idx == i1, -jnp.inf, logits)
        m2 = jnp.max(l2, axis=-1, keepdims=True)
        i2 = jnp.min(jnp.where(l2 == m2, idx, n_e), axis=-1, keepdims=True)
        top2 = (idx == i1) | (idx == i2)
        w = jnp.where(top2, jnp.exp(logits - m1), 0.0)
        cw_ref[...] = w / jnp.sum(w, axis=-1, keepdims=True)
        out_ref[:, 0, :] = jnp.zeros(out_ref.shape[::2], out_ref.dtype)

    xb = x.astype(jnp.bfloat16)
    g = jnp.dot(xb, gate_ref[0].astype(jnp.bfloat16),
                preferred_element_type=jnp.float32)
    u = jnp.dot(xb, up_ref[0].astype(jnp.bfloat16),
                preferred_element_type=jnp.float32)
    act = g * jax.nn.sigmoid(_SWIGLU_SCALE * g) * u
    lane = jax.lax.broadcasted_iota(jnp.int32, cw_ref.shape, 1)
    w_e = jnp.sum(jnp.where(lane == e, cw_ref[...], 0.0), axis=-1, keepdims=True)
    out_ref[:, 0, :] += jnp.dot((act * w_e).astype(jnp.bfloat16),
                                down_ref[0].astype(jnp.bfloat16),
                                preferred_element_type=jnp.float32)


def kernel(hidden_states, router_weight, gate_proj, up_proj, down_proj):
    b, s, h = hidden_states.shape
    e, _, f = gate_proj.shape
    nf = f // _FB

    return pl.pallas_call(
        _moe_body,
        grid=(e, nf),
        in_specs=[
            pl.BlockSpec((b, s, h), lambda ei, fi: (0, 0, 0)),
            pl.BlockSpec((e, h), lambda ei, fi: (0, 0)),
            pl.BlockSpec((1, h, _FB), lambda ei, fi: (ei, 0, fi)),
            pl.BlockSpec((1, h, _FB), lambda ei, fi: (ei, 0, fi)),
            pl.BlockSpec((1, _FB, h), lambda ei, fi: (ei, fi, 0)),
        ],
        out_specs=pl.BlockSpec((b, s, h), lambda ei, fi: (0, 0, 0)),
        out_shape=jax.ShapeDtypeStruct((b, s, h), jnp.float32),
        scratch_shapes=[pltpu.VMEM((b * s, e), jnp.float32)],
        compiler_params=pltpu.CompilerParams(
            dimension_semantics=("arbitrary", "arbitrary"),
        ),
    )(hidden_states, router_weight.T, gate_proj, up_proj, down_proj)
